# Initial kernel scaffold; baseline (speedup 1.0000x reference)
import jax
import jax.numpy as jnp
from jax.experimental import pallas as pl


def kernel(x, edge_indices, edge_attrs, We1, be1, We2, be2, We3, be3, eg, eb, Wn1, bn1, Wn2, bn2, Wn3, bn3, ng, nb):
    raise NotImplementedError("write your pallas kernel here")


# speedup vs baseline: 1.0426x; 1.0426x over previous
import functools

import jax
import jax.numpy as jnp
from jax import lax
from jax.experimental import pallas as pl
from jax.experimental.pallas import tpu as pltpu

D = 128
LN_EPS = 1e-5


def _ln_res(base, o, g, bb):
    mu = jnp.mean(o, axis=-1, keepdims=True)
    var = jnp.mean((o - mu) ** 2, axis=-1, keepdims=True)
    return base + g * (o - mu) * lax.rsqrt(var + LN_EPS) + bb


def _edge_body(ps_ref, qs_ref, e_ref, w1c, w2, b2, w3, b3, g, bb, out_ref):
    e = e_ref[...]
    h = ps_ref[...] + qs_ref[...] + jnp.dot(e, w1c[...], preferred_element_type=jnp.float32)
    h = jnp.maximum(h, 0.0)
    h = jnp.maximum(jnp.dot(h, w2[...], preferred_element_type=jnp.float32) + b2[...], 0.0)
    o = jnp.dot(h, w3[...], preferred_element_type=jnp.float32) + b3[...]
    out_ref[...] = _ln_res(e, o, g[...], bb[...])


def _node_body(x_ref, a_ref, w1a, w1b, b1, w2, b2, w3, b3, g, bb,
               p1a, p1b, pb1, x_out, p_out, q_out):
    x = x_ref[...]
    h = (jnp.dot(x, w1a[...], preferred_element_type=jnp.float32)
         + jnp.dot(a_ref[...], w1b[...], preferred_element_type=jnp.float32) + b1[...])
    h = jnp.maximum(h, 0.0)
    h = jnp.maximum(jnp.dot(h, w2[...], preferred_element_type=jnp.float32) + b2[...], 0.0)
    o = jnp.dot(h, w3[...], preferred_element_type=jnp.float32) + b3[...]
    xn = _ln_res(x, o, g[...], bb[...])
    x_out[...] = xn
    p_out[...] = jnp.dot(xn, p1a[...], preferred_element_type=jnp.float32) + pb1[...]
    q_out[...] = jnp.dot(xn, p1b[...], preferred_element_type=jnp.float32)


def _proj_body(x_ref, w1a, w1b, b1, p_out, q_out):
    x = x_ref[...]
    p_out[...] = jnp.dot(x, w1a[...], preferred_element_type=jnp.float32) + b1[...]
    q_out[...] = jnp.dot(x, w1b[...], preferred_element_type=jnp.float32)


def _full(shape):
    return pl.BlockSpec(shape, lambda i: (0,) * len(shape))


def _rows(block):
    return pl.BlockSpec((block, D), lambda i: (i, 0))


def _edge_call(E, BE):
    grid = E // BE
    w = _full((D, D))
    v = _full((1, D))
    return pl.pallas_call(
        _edge_body,
        grid=(grid,),
        in_specs=[_rows(BE), _rows(BE), _rows(BE), w, w, v, w, v, v, v],
        out_specs=_rows(BE),
        out_shape=jax.ShapeDtypeStruct((E, D), jnp.float32),
    )


def _node_call(N, BN):
    grid = N // BN
    w = _full((D, D))
    v = _full((1, D))
    out = jax.ShapeDtypeStruct((N, D), jnp.float32)
    return pl.pallas_call(
        _node_body,
        grid=(grid,),
        in_specs=[_rows(BN), _rows(BN), w, w, v, w, v, w, v, v, v, w, w, v],
        out_specs=[_rows(BN), _rows(BN), _rows(BN)],
        out_shape=[out, out, out],
    )


def _proj_call(N, BN):
    grid = N // BN
    w = _full((D, D))
    v = _full((1, D))
    out = jax.ShapeDtypeStruct((N, D), jnp.float32)
    return pl.pallas_call(
        _proj_body,
        grid=(grid,),
        in_specs=[_rows(BN), w, w, v],
        out_specs=[_rows(BN), _rows(BN)],
        out_shape=[out, out],
    )


def kernel(x, edge_indices, edge_attrs, We1, be1, We2, be2, We3, be3, eg, eb,
           Wn1, bn1, Wn2, bn2, Wn3, bn3, ng, nb):
    N, _ = x.shape
    E = edge_attrs.shape[1]
    T = We1.shape[0]
    row = edge_indices[0, 0]
    col = edge_indices[0, 1]
    e = edge_attrs[0]

    BE, BN = 2000, 2000
    edge_fn = _edge_call(E, BE)
    node_fn = _node_call(N, BN)
    proj_fn = _proj_call(N, BN)

    r2 = lambda a: a.reshape(1, D)

    p, q = proj_fn(x, We1[0, :D], We1[0, D:2 * D], r2(be1[0]))
    for t in range(T):
        ps = jnp.take(p, row, axis=0)
        qs = jnp.take(q, col, axis=0)
        e = edge_fn(ps, qs, e, We1[t, 2 * D:], We2[t], r2(be2[t]), We3[t],
                    r2(be3[t]), r2(eg[t]), r2(eb[t]))
        agg = jax.ops.segment_sum(e, col, num_segments=N)
        tn = (t + 1) % T
        x, p, q = node_fn(x, agg, Wn1[t, :D], Wn1[t, D:], r2(bn1[t]), Wn2[t],
                          r2(bn2[t]), Wn3[t], r2(bn3[t]), r2(ng[t]), r2(nb[t]),
                          We1[tn, :D], We1[tn, D:2 * D], r2(be1[tn]))
    return (x, e)


# speedup vs baseline: 3.0880x; 2.9620x over previous
import functools

import jax
import jax.numpy as jnp
from jax import lax
from jax.experimental import pallas as pl
from jax.experimental.pallas import tpu as pltpu
from jax.experimental.pallas import tpu_sc as plsc

D = 128
LN_EPS = 1e-5
_NC, _NS = 2, 16
_CH = 128


def _gather_call(N, E):
    NW = _NC * _NS
    EW = E // NW
    nfull, rem = EW // _CH, EW % _CH
    mesh = plsc.VectorSubcoreMesh(core_axis_name="c", subcore_axis_name="s")
    out = jax.ShapeDtypeStruct((E, D), jnp.float32)

    @functools.partial(
        pl.kernel, mesh=mesh, out_type=[out, out],
        scratch_types=[
            pltpu.VMEM((_CH,), jnp.int32), pltpu.VMEM((_CH,), jnp.int32),
            pltpu.VMEM((_CH, D), jnp.float32), pltpu.VMEM((_CH, D), jnp.float32),
            pltpu.VMEM((max(rem, 8),), jnp.int32),
            pltpu.VMEM((max(rem, 8),), jnp.int32),
            pltpu.VMEM((max(rem, 8), D), jnp.float32),
            pltpu.VMEM((max(rem, 8), D), jnp.float32),
            pltpu.SemaphoreType.DMA, pltpu.SemaphoreType.DMA,
        ],
    )
    def k(p_hbm, q_hbm, row_hbm, col_hbm, ps_hbm, qs_hbm,
          ia, ib, ba, bb_, ra, rb, rba, rbb, s1, s2):
        wid = lax.axis_index("s") * _NC + lax.axis_index("c")
        base = wid * EW

        def do(off, icur_a, icur_b, buf_a, buf_b, size):
            pltpu.sync_copy(row_hbm.at[pl.ds(base + off, size)], icur_a)
            pltpu.sync_copy(col_hbm.at[pl.ds(base + off, size)], icur_b)
            c1 = pltpu.async_copy(p_hbm.at[icur_a], buf_a, s1)
            c2 = pltpu.async_copy(q_hbm.at[icur_b], buf_b, s2)
            c1.wait()
            c2.wait()
            pltpu.sync_copy(buf_a, ps_hbm.at[pl.ds(base + off, size)])
            pltpu.sync_copy(buf_b, qs_hbm.at[pl.ds(base + off, size)])

        for ci in range(nfull):
            do(ci * _CH, ia, ib, ba, bb_, _CH)
        if rem:
            do(nfull * _CH, ra, rb, rba, rbb, rem)

    return k


def _scatter_call(N, E):
    NW = _NC * _NS
    EW = E // NW
    nfull, rem = EW // _CH, EW % _CH
    NR = -(-N // _NS) // 8 * 8
    NR_LAST = N - (_NS - 1) * NR
    mesh = plsc.VectorSubcoreMesh(core_axis_name="c", subcore_axis_name="s")

    @functools.partial(
        pl.kernel, mesh=mesh,
        out_type=jax.ShapeDtypeStruct((_NC, N, D), jnp.float32),
        scratch_types=[
            pltpu.VMEM((_CH,), jnp.int32), pltpu.VMEM((_CH, D), jnp.float32),
            pltpu.VMEM((max(rem, 8),), jnp.int32),
            pltpu.VMEM((max(rem, 8), D), jnp.float32),
            pltpu.VMEM_SHARED((N, D), jnp.float32),
        ],
    )
    def k(e_hbm, col_hbm, zero_hbm, out_hbm, idx, buf, ri, rbuf, acc):
        cid = lax.axis_index("c")
        sid = lax.axis_index("s")
        wid = sid * _NC + cid
        base = wid * EW
        @pl.when(sid < _NS - 1)
        def _():
            pltpu.sync_copy(zero_hbm.at[pl.ds(sid * NR, NR)],
                            acc.at[pl.ds(sid * NR, NR)])

        @pl.when(sid == _NS - 1)
        def _():
            pltpu.sync_copy(zero_hbm.at[pl.ds((_NS - 1) * NR, NR_LAST)],
                            acc.at[pl.ds((_NS - 1) * NR, NR_LAST)])

        plsc.subcore_barrier()

        def do(off, icur, bcur, size):
            pltpu.sync_copy(col_hbm.at[pl.ds(base + off, size)], icur)
            pltpu.sync_copy(e_hbm.at[pl.ds(base + off, size)], bcur)
            pltpu.sync_copy(bcur, acc.at[icur], add=True)

        for ci in range(nfull):
            do(ci * _CH, idx, buf, _CH)
        if rem:
            do(nfull * _CH, ri, rbuf, rem)
        plsc.subcore_barrier()

        @pl.when(sid < _NS - 1)
        def _():
            pltpu.sync_copy(acc.at[pl.ds(sid * NR, NR)],
                            out_hbm.at[cid, pl.ds(sid * NR, NR)])

        @pl.when(sid == _NS - 1)
        def _():
            pltpu.sync_copy(acc.at[pl.ds((_NS - 1) * NR, NR_LAST)],
                            out_hbm.at[cid, pl.ds((_NS - 1) * NR, NR_LAST)])

    return k


def _ln_res(base, o, g, bb):
    mu = jnp.mean(o, axis=-1, keepdims=True)
    var = jnp.mean((o - mu) ** 2, axis=-1, keepdims=True)
    return base + g * (o - mu) * lax.rsqrt(var + LN_EPS) + bb


def _edge_body(ps_ref, qs_ref, e_ref, w1c, w2, b2, w3, b3, g, bb, out_ref):
    e = e_ref[...]
    h = ps_ref[...] + qs_ref[...] + jnp.dot(e, w1c[...], preferred_element_type=jnp.float32)
    h = jnp.maximum(h, 0.0)
    h = jnp.maximum(jnp.dot(h, w2[...], preferred_element_type=jnp.float32) + b2[...], 0.0)
    o = jnp.dot(h, w3[...], preferred_element_type=jnp.float32) + b3[...]
    out_ref[...] = _ln_res(e, o, g[...], bb[...])


def _node_body(x_ref, a0_ref, a1_ref, w1a, w1b, b1, w2, b2, w3, b3, g, bb,
               p1a, p1b, pb1, x_out, p_out, q_out):
    x = x_ref[...]
    agg = a0_ref[...] + a1_ref[...]
    h = (jnp.dot(x, w1a[...], preferred_element_type=jnp.float32)
         + jnp.dot(agg, w1b[...], preferred_element_type=jnp.float32) + b1[...])
    h = jnp.maximum(h, 0.0)
    h = jnp.maximum(jnp.dot(h, w2[...], preferred_element_type=jnp.float32) + b2[...], 0.0)
    o = jnp.dot(h, w3[...], preferred_element_type=jnp.float32) + b3[...]
    xn = _ln_res(x, o, g[...], bb[...])
    x_out[...] = xn
    p_out[...] = jnp.dot(xn, p1a[...], preferred_element_type=jnp.float32) + pb1[...]
    q_out[...] = jnp.dot(xn, p1b[...], preferred_element_type=jnp.float32)


def _proj_body(x_ref, w1a, w1b, b1, p_out, q_out):
    x = x_ref[...]
    p_out[...] = jnp.dot(x, w1a[...], preferred_element_type=jnp.float32) + b1[...]
    q_out[...] = jnp.dot(x, w1b[...], preferred_element_type=jnp.float32)


def _full(shape):
    return pl.BlockSpec(shape, lambda i: (0,) * len(shape))


def _rows(block):
    return pl.BlockSpec((block, D), lambda i: (i, 0))


def _edge_call(E, BE):
    grid = E // BE
    w = _full((D, D))
    v = _full((1, D))
    return pl.pallas_call(
        _edge_body,
        grid=(grid,),
        in_specs=[_rows(BE), _rows(BE), _rows(BE), w, w, v, w, v, v, v],
        out_specs=_rows(BE),
        out_shape=jax.ShapeDtypeStruct((E, D), jnp.float32),
    )


def _node_call(N, BN):
    grid = N // BN
    w = _full((D, D))
    v = _full((1, D))
    out = jax.ShapeDtypeStruct((N, D), jnp.float32)
    return pl.pallas_call(
        _node_body,
        grid=(grid,),
        in_specs=[_rows(BN), _rows(BN), _rows(BN), w, w, v, w, v, w, v, v, v, w, w, v],
        out_specs=[_rows(BN), _rows(BN), _rows(BN)],
        out_shape=[out, out, out],
    )


def _proj_call(N, BN):
    grid = N // BN
    w = _full((D, D))
    v = _full((1, D))
    out = jax.ShapeDtypeStruct((N, D), jnp.float32)
    return pl.pallas_call(
        _proj_body,
        grid=(grid,),
        in_specs=[_rows(BN), w, w, v],
        out_specs=[_rows(BN), _rows(BN)],
        out_shape=[out, out],
    )


def kernel(x, edge_indices, edge_attrs, We1, be1, We2, be2, We3, be3, eg, eb,
           Wn1, bn1, Wn2, bn2, Wn3, bn3, ng, nb):
    N, _ = x.shape
    E = edge_attrs.shape[1]
    T = We1.shape[0]
    row = edge_indices[0, 0]
    col = edge_indices[0, 1]
    e = edge_attrs[0]

    BE, BN = 2000, 2000
    edge_fn = _edge_call(E, BE)
    node_fn = _node_call(N, BN)
    proj_fn = _proj_call(N, BN)
    gather_fn = _gather_call(N, E)
    scatter_fn = _scatter_call(N, E)

    r2 = lambda a: a.reshape(1, D)
    zeros_nd = jnp.zeros((N, D), jnp.float32)

    p, q = proj_fn(x, We1[0, :D], We1[0, D:2 * D], r2(be1[0]))
    for t in range(T):
        ps, qs = gather_fn(p, q, row, col)
        e = edge_fn(ps, qs, e, We1[t, 2 * D:], We2[t], r2(be2[t]), We3[t],
                    r2(be3[t]), r2(eg[t]), r2(eb[t]))
        parts = scatter_fn(e, col, zeros_nd)
        tn = (t + 1) % T
        x, p, q = node_fn(x, parts[0], parts[1], Wn1[t, :D], Wn1[t, D:],
                          r2(bn1[t]), Wn2[t], r2(bn2[t]), Wn3[t], r2(bn3[t]),
                          r2(ng[t]), r2(nb[t]),
                          We1[tn, :D], We1[tn, D:2 * D], r2(be1[tn]))
    return (x, e)


# speedup vs baseline: 3.5536x; 1.1507x over previous
import functools

import jax
import jax.numpy as jnp
from jax import lax
from jax.experimental import pallas as pl
from jax.experimental.pallas import tpu as pltpu
from jax.experimental.pallas import tpu_sc as plsc

D = 128
LN_EPS = 1e-5
_NC, _NS = 2, 16
_CH = 128


def _gather_call(N, E):
    NW = _NC * _NS
    EW = E // NW
    nfull, rem = EW // _CH, EW % _CH
    mesh = plsc.VectorSubcoreMesh(core_axis_name="c", subcore_axis_name="s")
    out = jax.ShapeDtypeStruct((E, D), jnp.float32)

    @functools.partial(
        pl.kernel, mesh=mesh, out_type=[out, out],
        scratch_types=[
            pltpu.VMEM((EW,), jnp.int32), pltpu.VMEM((EW,), jnp.int32),
            [pltpu.VMEM((_CH, D), jnp.float32) for _ in range(2)],
            [pltpu.VMEM((_CH, D), jnp.float32) for _ in range(2)],
            [pltpu.SemaphoreType.DMA for _ in range(4)],
            [pltpu.SemaphoreType.DMA for _ in range(4)],
            pltpu.VMEM((max(rem, 8), D), jnp.float32),
            pltpu.VMEM((max(rem, 8), D), jnp.float32),
        ],
    )
    def k(p_hbm, q_hbm, row_hbm, col_hbm, ps_hbm, qs_hbm,
          ir, ic, ba, bb_, gs, ws, rba, rbb):
        wid = lax.axis_index("s") * _NC + lax.axis_index("c")
        base = wid * EW
        pltpu.sync_copy(row_hbm.at[pl.ds(base, EW)], ir)
        pltpu.sync_copy(col_hbm.at[pl.ds(base, EW)], ic)

        wb = [None, None, None, None]

        def issue_gather(ci, slot):
            sl = pl.ds(ci * _CH, _CH)
            g1 = pltpu.async_copy(p_hbm.at[ir.at[sl]], ba[slot], gs[2 * slot])
            g2 = pltpu.async_copy(q_hbm.at[ic.at[sl]], bb_[slot], gs[2 * slot + 1])
            return g1, g2

        def issue_wb(ci, slot, g1, g2):
            g1.wait()
            g2.wait()
            sl = pl.ds(base + ci * _CH, _CH)
            wb[2 * slot] = pltpu.async_copy(ba[slot], ps_hbm.at[sl], ws[2 * slot])
            wb[2 * slot + 1] = pltpu.async_copy(bb_[slot], qs_hbm.at[sl],
                                                ws[2 * slot + 1])

        prev = None
        for ci in range(nfull):
            slot = ci % 2
            if wb[2 * slot] is not None:
                wb[2 * slot].wait()
                wb[2 * slot + 1].wait()
                wb[2 * slot] = wb[2 * slot + 1] = None
            cur = issue_gather(ci, slot)
            if prev is not None:
                issue_wb(ci - 1, 1 - slot, *prev)
            prev = cur
        issue_wb(nfull - 1, (nfull - 1) % 2, *prev)
        if rem:
            off = nfull * _CH
            sl = pl.ds(off, rem)
            c1 = pltpu.async_copy(p_hbm.at[ir.at[sl]], rba, gs[0])
            c2 = pltpu.async_copy(q_hbm.at[ic.at[sl]], rbb, gs[1])
            c1.wait()
            c2.wait()
            pltpu.sync_copy(rba, ps_hbm.at[pl.ds(base + off, rem)])
            pltpu.sync_copy(rbb, qs_hbm.at[pl.ds(base + off, rem)])
        for d in wb:
            if d is not None:
                d.wait()

    return k


def _scatter_call(N, E):
    NW = _NC * _NS
    EW = E // NW
    nfull, rem = EW // _CH, EW % _CH
    NR = -(-N // _NS) // 8 * 8
    NR_LAST = N - (_NS - 1) * NR
    mesh = plsc.VectorSubcoreMesh(core_axis_name="c", subcore_axis_name="s")

    @functools.partial(
        pl.kernel, mesh=mesh,
        out_type=jax.ShapeDtypeStruct((_NC, N, D), jnp.float32),
        scratch_types=[
            pltpu.VMEM((_CH,), jnp.int32), pltpu.VMEM((_CH, D), jnp.float32),
            pltpu.VMEM((max(rem, 8),), jnp.int32),
            pltpu.VMEM((max(rem, 8), D), jnp.float32),
            pltpu.VMEM_SHARED((N, D), jnp.float32),
        ],
    )
    def k(e_hbm, col_hbm, zero_hbm, out_hbm, idx, buf, ri, rbuf, acc):
        cid = lax.axis_index("c")
        sid = lax.axis_index("s")
        wid = sid * _NC + cid
        base = wid * EW
        @pl.when(sid < _NS - 1)
        def _():
            pltpu.sync_copy(zero_hbm.at[pl.ds(sid * NR, NR)],
                            acc.at[pl.ds(sid * NR, NR)])

        @pl.when(sid == _NS - 1)
        def _():
            pltpu.sync_copy(zero_hbm.at[pl.ds((_NS - 1) * NR, NR_LAST)],
                            acc.at[pl.ds((_NS - 1) * NR, NR_LAST)])

        plsc.subcore_barrier()

        def do(off, icur, bcur, size):
            pltpu.sync_copy(col_hbm.at[pl.ds(base + off, size)], icur)
            pltpu.sync_copy(e_hbm.at[pl.ds(base + off, size)], bcur)
            pltpu.sync_copy(bcur, acc.at[icur], add=True)

        for ci in range(nfull):
            do(ci * _CH, idx, buf, _CH)
        if rem:
            do(nfull * _CH, ri, rbuf, rem)
        plsc.subcore_barrier()

        @pl.when(sid < _NS - 1)
        def _():
            pltpu.sync_copy(acc.at[pl.ds(sid * NR, NR)],
                            out_hbm.at[cid, pl.ds(sid * NR, NR)])

        @pl.when(sid == _NS - 1)
        def _():
            pltpu.sync_copy(acc.at[pl.ds((_NS - 1) * NR, NR_LAST)],
                            out_hbm.at[cid, pl.ds((_NS - 1) * NR, NR_LAST)])

    return k


def _ln_res(base, o, g, bb):
    mu = jnp.mean(o, axis=-1, keepdims=True)
    var = jnp.mean((o - mu) ** 2, axis=-1, keepdims=True)
    return base + g * (o - mu) * lax.rsqrt(var + LN_EPS) + bb


def _edge_body(ps_ref, qs_ref, e_ref, w1c, w2, b2, w3, b3, g, bb, out_ref):
    e = e_ref[...]
    pq = ps_ref[...].astype(jnp.float32) + qs_ref[...].astype(jnp.float32)
    h = pq + jnp.dot(e, w1c[...], preferred_element_type=jnp.float32)
    h = jnp.maximum(h, 0.0)
    h = jnp.maximum(jnp.dot(h, w2[...], preferred_element_type=jnp.float32) + b2[...], 0.0)
    o = jnp.dot(h, w3[...], preferred_element_type=jnp.float32) + b3[...]
    out_ref[...] = _ln_res(e, o, g[...], bb[...])


def _node_body(x_ref, a0_ref, a1_ref, w1a, w1b, b1, w2, b2, w3, b3, g, bb,
               p1a, p1b, pb1, x_out, p_out, q_out):
    x = x_ref[...]
    agg = a0_ref[...] + a1_ref[...]
    h = (jnp.dot(x, w1a[...], preferred_element_type=jnp.float32)
         + jnp.dot(agg, w1b[...], preferred_element_type=jnp.float32) + b1[...])
    h = jnp.maximum(h, 0.0)
    h = jnp.maximum(jnp.dot(h, w2[...], preferred_element_type=jnp.float32) + b2[...], 0.0)
    o = jnp.dot(h, w3[...], preferred_element_type=jnp.float32) + b3[...]
    xn = _ln_res(x, o, g[...], bb[...])
    x_out[...] = xn
    p_out[...] = jnp.dot(xn, p1a[...], preferred_element_type=jnp.float32) + pb1[...]
    q_out[...] = jnp.dot(xn, p1b[...], preferred_element_type=jnp.float32)


def _proj_body(x_ref, w1a, w1b, b1, p_out, q_out):
    x = x_ref[...]
    p_out[...] = jnp.dot(x, w1a[...], preferred_element_type=jnp.float32) + b1[...]
    q_out[...] = jnp.dot(x, w1b[...], preferred_element_type=jnp.float32)


def _full(shape):
    return pl.BlockSpec(shape, lambda i: (0,) * len(shape))


def _rows(block):
    return pl.BlockSpec((block, D), lambda i: (i, 0))


def _edge_call(E, BE):
    grid = E // BE
    w = _full((D, D))
    v = _full((1, D))
    return pl.pallas_call(
        _edge_body,
        grid=(grid,),
        in_specs=[_rows(BE), _rows(BE), _rows(BE), w, w, v, w, v, v, v],
        out_specs=_rows(BE),
        out_shape=jax.ShapeDtypeStruct((E, D), jnp.float32),
    )


def _node_call(N, BN):
    grid = N // BN
    w = _full((D, D))
    v = _full((1, D))
    out = jax.ShapeDtypeStruct((N, D), jnp.float32)
    outh = jax.ShapeDtypeStruct((N, D), jnp.bfloat16)
    return pl.pallas_call(
        _node_body,
        grid=(grid,),
        in_specs=[_rows(BN), _rows(BN), _rows(BN), w, w, v, w, v, w, v, v, v, w, w, v],
        out_specs=[_rows(BN), _rows(BN), _rows(BN)],
        out_shape=[out, out, out],
    )


def _proj_call(N, BN):
    grid = N // BN
    w = _full((D, D))
    v = _full((1, D))
    out = jax.ShapeDtypeStruct((N, D), jnp.float32)
    return pl.pallas_call(
        _proj_body,
        grid=(grid,),
        in_specs=[_rows(BN), w, w, v],
        out_specs=[_rows(BN), _rows(BN)],
        out_shape=[out, out],
    )


def kernel(x, edge_indices, edge_attrs, We1, be1, We2, be2, We3, be3, eg, eb,
           Wn1, bn1, Wn2, bn2, Wn3, bn3, ng, nb):
    N, _ = x.shape
    E = edge_attrs.shape[1]
    T = We1.shape[0]
    row = edge_indices[0, 0]
    col = edge_indices[0, 1]
    e = edge_attrs[0]

    BE, BN = 2000, 2000
    edge_fn = _edge_call(E, BE)
    node_fn = _node_call(N, BN)
    proj_fn = _proj_call(N, BN)
    gather_fn = _gather_call(N, E)
    scatter_fn = _scatter_call(N, E)

    r2 = lambda a: a.reshape(1, D)
    zeros_nd = jnp.zeros((N, D), jnp.float32)

    p, q = proj_fn(x, We1[0, :D], We1[0, D:2 * D], r2(be1[0]))
    for t in range(T):
        ps, qs = gather_fn(p, q, row, col)
        e = edge_fn(ps, qs, e, We1[t, 2 * D:], We2[t], r2(be2[t]), We3[t],
                    r2(be3[t]), r2(eg[t]), r2(eb[t]))
        parts = scatter_fn(e, col, zeros_nd)
        tn = (t + 1) % T
        x, p, q = node_fn(x, parts[0], parts[1], Wn1[t, :D], Wn1[t, D:],
                          r2(bn1[t]), Wn2[t], r2(bn2[t]), Wn3[t], r2(bn3[t]),
                          r2(ng[t]), r2(nb[t]),
                          We1[tn, :D], We1[tn, D:2 * D], r2(be1[tn]))
    return (x, e)


# speedup vs baseline: 4.0540x; 1.1408x over previous
import functools

import jax
import jax.numpy as jnp
from jax import lax
from jax.experimental import pallas as pl
from jax.experimental.pallas import tpu as pltpu
from jax.experimental.pallas import tpu_sc as plsc

D = 128
LN_EPS = 1e-5
_NC, _NS = 2, 16
_CH = 128


def _gather_call(N, E):
    NW = _NC * _NS
    EW = E // NW
    nfull, rem = EW // _CH, EW % _CH
    mesh = plsc.VectorSubcoreMesh(core_axis_name="c", subcore_axis_name="s")
    out = jax.ShapeDtypeStruct((E, D), jnp.float32)

    @functools.partial(
        pl.kernel, mesh=mesh, out_type=[out, out],
        scratch_types=[
            pltpu.VMEM((EW,), jnp.int32), pltpu.VMEM((EW,), jnp.int32),
            [pltpu.VMEM((_CH, D), jnp.float32) for _ in range(2)],
            [pltpu.VMEM((_CH, D), jnp.float32) for _ in range(2)],
            [pltpu.SemaphoreType.DMA for _ in range(4)],
            [pltpu.SemaphoreType.DMA for _ in range(4)],
            pltpu.VMEM((max(rem, 8), D), jnp.float32),
            pltpu.VMEM((max(rem, 8), D), jnp.float32),
        ],
    )
    def k(p_hbm, q_hbm, row_hbm, col_hbm, ps_hbm, qs_hbm,
          ir, ic, ba, bb_, gs, ws, rba, rbb):
        wid = lax.axis_index("s") * _NC + lax.axis_index("c")
        base = wid * EW
        pltpu.sync_copy(row_hbm.at[pl.ds(base, EW)], ir)
        pltpu.sync_copy(col_hbm.at[pl.ds(base, EW)], ic)

        wb = [None, None, None, None]

        def issue_gather(ci, slot):
            sl = pl.ds(ci * _CH, _CH)
            g1 = pltpu.async_copy(p_hbm.at[ir.at[sl]], ba[slot], gs[2 * slot])
            g2 = pltpu.async_copy(q_hbm.at[ic.at[sl]], bb_[slot], gs[2 * slot + 1])
            return g1, g2

        def issue_wb(ci, slot, g1, g2):
            g1.wait()
            g2.wait()
            sl = pl.ds(base + ci * _CH, _CH)
            wb[2 * slot] = pltpu.async_copy(ba[slot], ps_hbm.at[sl], ws[2 * slot])
            wb[2 * slot + 1] = pltpu.async_copy(bb_[slot], qs_hbm.at[sl],
                                                ws[2 * slot + 1])

        prev = None
        for ci in range(nfull):
            slot = ci % 2
            if wb[2 * slot] is not None:
                wb[2 * slot].wait()
                wb[2 * slot + 1].wait()
                wb[2 * slot] = wb[2 * slot + 1] = None
            cur = issue_gather(ci, slot)
            if prev is not None:
                issue_wb(ci - 1, 1 - slot, *prev)
            prev = cur
        issue_wb(nfull - 1, (nfull - 1) % 2, *prev)
        if rem:
            off = nfull * _CH
            sl = pl.ds(off, rem)
            c1 = pltpu.async_copy(p_hbm.at[ir.at[sl]], rba, gs[0])
            c2 = pltpu.async_copy(q_hbm.at[ic.at[sl]], rbb, gs[1])
            c1.wait()
            c2.wait()
            pltpu.sync_copy(rba, ps_hbm.at[pl.ds(base + off, rem)])
            pltpu.sync_copy(rbb, qs_hbm.at[pl.ds(base + off, rem)])
        for d in wb:
            if d is not None:
                d.wait()

    return k


def _scatter_call(N, E):
    NW = _NC * _NS
    EW = E // NW
    nfull, rem = EW // _CH, EW % _CH
    NR = -(-N // _NS) // 8 * 8
    NR_LAST = N - (_NS - 1) * NR
    mesh = plsc.VectorSubcoreMesh(core_axis_name="c", subcore_axis_name="s")

    @functools.partial(
        pl.kernel, mesh=mesh,
        out_type=jax.ShapeDtypeStruct((_NC, N, D), jnp.float32),
        scratch_types=[
            [pltpu.VMEM((_CH,), jnp.int32) for _ in range(2)],
            [pltpu.VMEM((_CH, D), jnp.float32) for _ in range(2)],
            pltpu.VMEM((max(rem, 8),), jnp.int32),
            pltpu.VMEM((max(rem, 8), D), jnp.float32),
            pltpu.VMEM_SHARED((N, D), jnp.float32),
            [pltpu.SemaphoreType.DMA for _ in range(2)],
            [pltpu.SemaphoreType.DMA for _ in range(2)],
        ],
    )
    def k(e_hbm, col_hbm, zero_hbm, out_hbm, idx2, buf2, ri, rbuf, acc, isem, lsem):
        cid = lax.axis_index("c")
        sid = lax.axis_index("s")
        wid = sid * _NC + cid
        base = wid * EW

        def load(ci, slot):
            sl = pl.ds(base + ci * _CH, _CH)
            i = pltpu.async_copy(col_hbm.at[sl], idx2[slot], isem[slot])
            e = pltpu.async_copy(e_hbm.at[sl], buf2[slot], lsem[slot])
            return i, e

        prev = load(0, 0)

        @pl.when(sid < _NS - 1)
        def _():
            pltpu.sync_copy(zero_hbm.at[pl.ds(sid * NR, NR)],
                            acc.at[pl.ds(sid * NR, NR)])

        @pl.when(sid == _NS - 1)
        def _():
            pltpu.sync_copy(zero_hbm.at[pl.ds((_NS - 1) * NR, NR_LAST)],
                            acc.at[pl.ds((_NS - 1) * NR, NR_LAST)])

        plsc.subcore_barrier()

        for ci in range(nfull):
            slot = ci % 2
            nxt = load(ci + 1, 1 - slot) if ci + 1 < nfull else None
            prev[0].wait()
            prev[1].wait()
            pltpu.sync_copy(buf2[slot], acc.at[idx2[slot]], add=True)
            prev = nxt
        if rem:
            off = nfull * _CH
            pltpu.sync_copy(col_hbm.at[pl.ds(base + off, rem)], ri)
            pltpu.sync_copy(e_hbm.at[pl.ds(base + off, rem)], rbuf)
            pltpu.sync_copy(rbuf, acc.at[ri], add=True)
        plsc.subcore_barrier()

        @pl.when(sid < _NS - 1)
        def _():
            pltpu.sync_copy(acc.at[pl.ds(sid * NR, NR)],
                            out_hbm.at[cid, pl.ds(sid * NR, NR)])

        @pl.when(sid == _NS - 1)
        def _():
            pltpu.sync_copy(acc.at[pl.ds((_NS - 1) * NR, NR_LAST)],
                            out_hbm.at[cid, pl.ds((_NS - 1) * NR, NR_LAST)])

    return k


def _ln_res(base, o, g, bb):
    mu = jnp.mean(o, axis=-1, keepdims=True)
    var = jnp.mean((o - mu) ** 2, axis=-1, keepdims=True)
    return base + g * (o - mu) * lax.rsqrt(var + LN_EPS) + bb


def _edge_body(ps_ref, qs_ref, e_ref, w1c, w2, b2, w3, b3, g, bb, out_ref):
    e = e_ref[...]
    pq = ps_ref[...].astype(jnp.float32) + qs_ref[...].astype(jnp.float32)
    h = pq + jnp.dot(e, w1c[...], preferred_element_type=jnp.float32)
    h = jnp.maximum(h, 0.0)
    h = jnp.maximum(jnp.dot(h, w2[...], preferred_element_type=jnp.float32) + b2[...], 0.0)
    o = jnp.dot(h, w3[...], preferred_element_type=jnp.float32) + b3[...]
    out_ref[...] = _ln_res(e, o, g[...], bb[...])


def _node_body(x_ref, a0_ref, a1_ref, w1a, w1b, b1, w2, b2, w3, b3, g, bb,
               p1a, p1b, pb1, x_out, p_out, q_out):
    x = x_ref[...]
    agg = a0_ref[...] + a1_ref[...]
    h = (jnp.dot(x, w1a[...], preferred_element_type=jnp.float32)
         + jnp.dot(agg, w1b[...], preferred_element_type=jnp.float32) + b1[...])
    h = jnp.maximum(h, 0.0)
    h = jnp.maximum(jnp.dot(h, w2[...], preferred_element_type=jnp.float32) + b2[...], 0.0)
    o = jnp.dot(h, w3[...], preferred_element_type=jnp.float32) + b3[...]
    xn = _ln_res(x, o, g[...], bb[...])
    x_out[...] = xn
    p_out[...] = jnp.dot(xn, p1a[...], preferred_element_type=jnp.float32) + pb1[...]
    q_out[...] = jnp.dot(xn, p1b[...], preferred_element_type=jnp.float32)


def _proj_body(x_ref, w1a, w1b, b1, p_out, q_out):
    x = x_ref[...]
    p_out[...] = jnp.dot(x, w1a[...], preferred_element_type=jnp.float32) + b1[...]
    q_out[...] = jnp.dot(x, w1b[...], preferred_element_type=jnp.float32)


def _full(shape):
    return pl.BlockSpec(shape, lambda i: (0,) * len(shape))


def _rows(block):
    return pl.BlockSpec((block, D), lambda i: (i, 0))


def _edge_call(E, BE):
    grid = E // BE
    w = _full((D, D))
    v = _full((1, D))
    return pl.pallas_call(
        _edge_body,
        grid=(grid,),
        in_specs=[_rows(BE), _rows(BE), _rows(BE), w, w, v, w, v, v, v],
        out_specs=_rows(BE),
        out_shape=jax.ShapeDtypeStruct((E, D), jnp.float32),
    )


def _node_call(N, BN):
    grid = N // BN
    w = _full((D, D))
    v = _full((1, D))
    out = jax.ShapeDtypeStruct((N, D), jnp.float32)
    outh = jax.ShapeDtypeStruct((N, D), jnp.bfloat16)
    return pl.pallas_call(
        _node_body,
        grid=(grid,),
        in_specs=[_rows(BN), _rows(BN), _rows(BN), w, w, v, w, v, w, v, v, v, w, w, v],
        out_specs=[_rows(BN), _rows(BN), _rows(BN)],
        out_shape=[out, out, out],
    )


def _proj_call(N, BN):
    grid = N // BN
    w = _full((D, D))
    v = _full((1, D))
    out = jax.ShapeDtypeStruct((N, D), jnp.float32)
    return pl.pallas_call(
        _proj_body,
        grid=(grid,),
        in_specs=[_rows(BN), w, w, v],
        out_specs=[_rows(BN), _rows(BN)],
        out_shape=[out, out],
    )


def kernel(x, edge_indices, edge_attrs, We1, be1, We2, be2, We3, be3, eg, eb,
           Wn1, bn1, Wn2, bn2, Wn3, bn3, ng, nb):
    N, _ = x.shape
    E = edge_attrs.shape[1]
    T = We1.shape[0]
    row = edge_indices[0, 0]
    col = edge_indices[0, 1]
    e = edge_attrs[0]

    BE, BN = 2000, 2000
    edge_fn = _edge_call(E, BE)
    node_fn = _node_call(N, BN)
    proj_fn = _proj_call(N, BN)
    gather_fn = _gather_call(N, E)
    scatter_fn = _scatter_call(N, E)

    r2 = lambda a: a.reshape(1, D)
    zeros_nd = jnp.zeros((N, D), jnp.float32)

    p, q = proj_fn(x, We1[0, :D], We1[0, D:2 * D], r2(be1[0]))
    for t in range(T):
        ps, qs = gather_fn(p, q, row, col)
        e = edge_fn(ps, qs, e, We1[t, 2 * D:], We2[t], r2(be2[t]), We3[t],
                    r2(be3[t]), r2(eg[t]), r2(eb[t]))
        parts = scatter_fn(e, col, zeros_nd)
        tn = (t + 1) % T
        x, p, q = node_fn(x, parts[0], parts[1], Wn1[t, :D], Wn1[t, D:],
                          r2(bn1[t]), Wn2[t], r2(bn2[t]), Wn3[t], r2(bn3[t]),
                          r2(ng[t]), r2(nb[t]),
                          We1[tn, :D], We1[tn, D:2 * D], r2(be1[tn]))
    return (x, e)


# speedup vs baseline: 5.5517x; 1.3695x over previous
import functools

import jax
import jax.numpy as jnp
from jax import lax
from jax.experimental import pallas as pl
from jax.experimental.pallas import tpu as pltpu
from jax.experimental.pallas import tpu_sc as plsc

D = 128
LN_EPS = 1e-5
_NC, _NS = 2, 16
_CH = 128


def _gather_call(N, E):
    NW = _NC * _NS
    EW = E // NW
    nfull, rem = EW // _CH, EW % _CH
    mesh = plsc.VectorSubcoreMesh(core_axis_name="c", subcore_axis_name="s")
    out = jax.ShapeDtypeStruct((E, D), jnp.float32)

    NBUF = 6

    @functools.partial(
        pl.kernel, mesh=mesh, out_type=out,
        scratch_types=[
            pltpu.VMEM((EW,), jnp.int32), pltpu.VMEM((EW,), jnp.int32),
            [pltpu.VMEM((_CH, D), jnp.float32) for _ in range(NBUF)],
            [pltpu.SemaphoreType.DMA for _ in range(NBUF)],
            [pltpu.SemaphoreType.DMA for _ in range(NBUF)],
            [pltpu.SemaphoreType.DMA for _ in range(NBUF)],
            pltpu.VMEM((max(rem, 8), D), jnp.float32),
        ],
    )
    def k(p_hbm, q_hbm, row_hbm, col_hbm, pq_hbm,
          ir, ic, ba, gs, hs, ws, rba):
        wid = lax.axis_index("s") * _NC + lax.axis_index("c")
        base = wid * EW
        pltpu.sync_copy(row_hbm.at[pl.ds(base, EW)], ir)
        pltpu.sync_copy(col_hbm.at[pl.ds(base, EW)], ic)

        g1d = [None] * nfull
        g2d = [None] * nfull
        wbd = [None] * NBUF

        def stage_g1(ci):
            slot = ci % NBUF
            if wbd[slot] is not None:
                wbd[slot].wait()
                wbd[slot] = None
            g1d[ci] = pltpu.async_copy(
                p_hbm.at[ir.at[pl.ds(ci * _CH, _CH)]], ba[slot], gs[slot])

        def stage_g2(ci):
            slot = ci % NBUF
            g1d[ci].wait()
            g2d[ci] = pltpu.async_copy(
                q_hbm.at[ic.at[pl.ds(ci * _CH, _CH)]], ba[slot], hs[slot],
                add=True)

        def stage_wb(ci):
            slot = ci % NBUF
            g2d[ci].wait()
            wbd[slot] = pltpu.async_copy(
                ba[slot], pq_hbm.at[pl.ds(base + ci * _CH, _CH)], ws[slot])

        for ci in range(nfull + 2):
            if ci < nfull:
                stage_g1(ci)
            if 1 <= ci and ci - 1 < nfull:
                stage_g2(ci - 1)
            if 2 <= ci and ci - 2 < nfull:
                stage_wb(ci - 2)
        if rem:
            off = nfull * _CH
            sl = pl.ds(off, rem)
            pltpu.async_copy(p_hbm.at[ir.at[sl]], rba, gs[0]).wait()
            pltpu.async_copy(q_hbm.at[ic.at[sl]], rba, hs[0], add=True).wait()
            pltpu.sync_copy(rba, pq_hbm.at[pl.ds(base + off, rem)])
        for d in wbd:
            if d is not None:
                d.wait()

    return k


def _scatter_call(N, E):
    NW = _NC * _NS
    EW = E // NW
    nfull, rem = EW // _CH, EW % _CH
    NR = -(-N // _NS) // 8 * 8
    NR_LAST = N - (_NS - 1) * NR
    mesh = plsc.VectorSubcoreMesh(core_axis_name="c", subcore_axis_name="s")

    @functools.partial(
        pl.kernel, mesh=mesh,
        out_type=[jax.ShapeDtypeStruct((N, D), jnp.float32)] * _NC,
        scratch_types=[
            [pltpu.VMEM((_CH,), jnp.int32) for _ in range(3)],
            [pltpu.VMEM((_CH, D), jnp.float32) for _ in range(3)],
            pltpu.VMEM((max(rem, 8),), jnp.int32),
            pltpu.VMEM((max(rem, 8), D), jnp.float32),
            pltpu.VMEM_SHARED((N, D), jnp.float32),
            [pltpu.SemaphoreType.DMA for _ in range(3)],
            [pltpu.SemaphoreType.DMA for _ in range(3)],
        ],
    )
    def k(e_hbm, col_hbm, zero_hbm, out0_hbm, out1_hbm, idx2, buf2, ri, rbuf,
          acc, isem, lsem):
        cid = lax.axis_index("c")
        sid = lax.axis_index("s")
        wid = sid * _NC + cid
        base = wid * EW

        def load(ci, slot):
            sl = pl.ds(base + ci * _CH, _CH)
            i = pltpu.async_copy(col_hbm.at[sl], idx2[slot], isem[slot])
            e = pltpu.async_copy(e_hbm.at[sl], buf2[slot], lsem[slot])
            return i, e

        prev = load(0, 0)

        @pl.when(sid < _NS - 1)
        def _():
            pltpu.sync_copy(zero_hbm.at[pl.ds(sid * NR, NR)],
                            acc.at[pl.ds(sid * NR, NR)])

        @pl.when(sid == _NS - 1)
        def _():
            pltpu.sync_copy(zero_hbm.at[pl.ds((_NS - 1) * NR, NR_LAST)],
                            acc.at[pl.ds((_NS - 1) * NR, NR_LAST)])

        plsc.subcore_barrier()

        pend = [prev, load(1, 1)]
        for ci in range(nfull):
            slot = ci % 3
            if ci + 2 < nfull:
                pend.append(load(ci + 2, (ci + 2) % 3))
            cur = pend.pop(0)
            cur[0].wait()
            cur[1].wait()
            pltpu.sync_copy(buf2[slot], acc.at[idx2[slot]], add=True)
        if rem:
            off = nfull * _CH
            pltpu.sync_copy(col_hbm.at[pl.ds(base + off, rem)], ri)
            pltpu.sync_copy(e_hbm.at[pl.ds(base + off, rem)], rbuf)
            pltpu.sync_copy(rbuf, acc.at[ri], add=True)
        plsc.subcore_barrier()
        for c, out_hbm in enumerate((out0_hbm, out1_hbm)):
            @pl.when(jnp.logical_and(cid == c, sid < _NS - 1))
            def _():
                pltpu.sync_copy(acc.at[pl.ds(sid * NR, NR)],
                                out_hbm.at[pl.ds(sid * NR, NR)])

            @pl.when(jnp.logical_and(cid == c, sid == _NS - 1))
            def _():
                pltpu.sync_copy(acc.at[pl.ds((_NS - 1) * NR, NR_LAST)],
                                out_hbm.at[pl.ds((_NS - 1) * NR, NR_LAST)])

    return k


def _ln_res(base, o, g, bb):
    mu = jnp.mean(o, axis=-1, keepdims=True)
    var = jnp.mean((o - mu) ** 2, axis=-1, keepdims=True)
    return base + g * (o - mu) * lax.rsqrt(var + LN_EPS) + bb


def _edge_body(pq_ref, e_ref, w1c, w2, b2, w3, b3, g, bb, out_ref):
    e = e_ref[...]
    h = pq_ref[...] + jnp.dot(e, w1c[...], preferred_element_type=jnp.float32)
    h = jnp.maximum(h, 0.0)
    h = jnp.maximum(jnp.dot(h, w2[...], preferred_element_type=jnp.float32) + b2[...], 0.0)
    o = jnp.dot(h, w3[...], preferred_element_type=jnp.float32) + b3[...]
    out_ref[...] = _ln_res(e, o, g[...], bb[...])


def _node_body(x_ref, a0_ref, a1_ref, w1a, w1b, b1, w2, b2, w3, b3, g, bb,
               p1a, p1b, pb1, x_out, p_out, q_out):
    x = x_ref[...]
    agg = a0_ref[...] + a1_ref[...]
    h = (jnp.dot(x, w1a[...], preferred_element_type=jnp.float32)
         + jnp.dot(agg, w1b[...], preferred_element_type=jnp.float32) + b1[...])
    h = jnp.maximum(h, 0.0)
    h = jnp.maximum(jnp.dot(h, w2[...], preferred_element_type=jnp.float32) + b2[...], 0.0)
    o = jnp.dot(h, w3[...], preferred_element_type=jnp.float32) + b3[...]
    xn = _ln_res(x, o, g[...], bb[...])
    x_out[...] = xn
    p_out[...] = jnp.dot(xn, p1a[...], preferred_element_type=jnp.float32) + pb1[...]
    q_out[...] = jnp.dot(xn, p1b[...], preferred_element_type=jnp.float32)


def _proj_body(x_ref, w1a, w1b, b1, p_out, q_out):
    x = x_ref[...]
    p_out[...] = jnp.dot(x, w1a[...], preferred_element_type=jnp.float32) + b1[...]
    q_out[...] = jnp.dot(x, w1b[...], preferred_element_type=jnp.float32)


def _full(shape):
    return pl.BlockSpec(shape, lambda i: (0,) * len(shape))


def _rows(block):
    return pl.BlockSpec((block, D), lambda i: (i, 0))


def _edge_call(E, BE):
    grid = E // BE
    w = _full((D, D))
    v = _full((1, D))
    return pl.pallas_call(
        _edge_body,
        grid=(grid,),
        in_specs=[_rows(BE), _rows(BE), w, w, v, w, v, v, v],
        out_specs=_rows(BE),
        out_shape=jax.ShapeDtypeStruct((E, D), jnp.float32),
    )


def _node_call(N, BN):
    grid = N // BN
    w = _full((D, D))
    v = _full((1, D))
    out = jax.ShapeDtypeStruct((N, D), jnp.float32)
    outh = jax.ShapeDtypeStruct((N, D), jnp.bfloat16)
    return pl.pallas_call(
        _node_body,
        grid=(grid,),
        in_specs=[_rows(BN), _rows(BN), _rows(BN), w, w, v, w, v, w, v, v, v, w, w, v],
        out_specs=[_rows(BN), _rows(BN), _rows(BN)],
        out_shape=[out, out, out],
    )


def _proj_call(N, BN):
    grid = N // BN
    w = _full((D, D))
    v = _full((1, D))
    out = jax.ShapeDtypeStruct((N, D), jnp.float32)
    return pl.pallas_call(
        _proj_body,
        grid=(grid,),
        in_specs=[_rows(BN), w, w, v],
        out_specs=[_rows(BN), _rows(BN)],
        out_shape=[out, out],
    )


def kernel(x, edge_indices, edge_attrs, We1, be1, We2, be2, We3, be3, eg, eb,
           Wn1, bn1, Wn2, bn2, Wn3, bn3, ng, nb):
    N, _ = x.shape
    E = edge_attrs.shape[1]
    T = We1.shape[0]
    row = edge_indices[0, 0]
    col = edge_indices[0, 1]
    e = edge_attrs[0]

    BE, BN = 8000, 2000
    edge_fn = _edge_call(E, BE)
    node_fn = _node_call(N, BN)
    proj_fn = _proj_call(N, BN)
    gather_fn = _gather_call(N, E)
    scatter_fn = _scatter_call(N, E)

    r2 = lambda a: a.reshape(1, D)
    zeros_nd = jnp.zeros((N, D), jnp.float32)

    p, q = proj_fn(x, We1[0, :D], We1[0, D:2 * D], r2(be1[0]))
    for t in range(T):
        pq = gather_fn(p, q, row, col)
        e = edge_fn(pq, e, We1[t, 2 * D:], We2[t], r2(be2[t]), We3[t],
                    r2(be3[t]), r2(eg[t]), r2(eb[t]))
        a0, a1 = scatter_fn(e, col, zeros_nd)
        tn = (t + 1) % T
        x, p, q = node_fn(x, a0, a1, Wn1[t, :D], Wn1[t, D:],
                          r2(bn1[t]), Wn2[t], r2(bn2[t]), Wn3[t], r2(bn3[t]),
                          r2(ng[t]), r2(nb[t]),
                          We1[tn, :D], We1[tn, D:2 * D], r2(be1[tn]))
    return (x, e)


# speedup vs baseline: 5.5769x; 1.0045x over previous
import functools

import jax
import jax.numpy as jnp
from jax import lax
from jax.experimental import pallas as pl
from jax.experimental.pallas import tpu as pltpu
from jax.experimental.pallas import tpu_sc as plsc

D = 128
LN_EPS = 1e-5
_NC, _NS = 2, 16
_CH = 128


def _gather_call(N, E):
    NW = _NC * _NS
    EW = E // NW
    nfull, rem = EW // _CH, EW % _CH
    mesh = plsc.VectorSubcoreMesh(core_axis_name="c", subcore_axis_name="s")
    out = jax.ShapeDtypeStruct((E, D), jnp.float32)

    NBUF = 6

    @functools.partial(
        pl.kernel, mesh=mesh, out_type=out,
        scratch_types=[
            pltpu.VMEM((EW,), jnp.int32), pltpu.VMEM((EW,), jnp.int32),
            [pltpu.VMEM((_CH, D), jnp.float32) for _ in range(NBUF)],
            [pltpu.SemaphoreType.DMA for _ in range(NBUF)],
            [pltpu.SemaphoreType.DMA for _ in range(NBUF)],
            [pltpu.SemaphoreType.DMA for _ in range(NBUF)],
            pltpu.VMEM((max(rem, 8), D), jnp.float32),
        ],
    )
    def k(p_hbm, q_hbm, row_hbm, col_hbm, pq_hbm,
          ir, ic, ba, gs, hs, ws, rba):
        wid = lax.axis_index("s") * _NC + lax.axis_index("c")
        base = wid * EW
        pltpu.sync_copy(row_hbm.at[pl.ds(base, EW)], ir)
        pltpu.sync_copy(col_hbm.at[pl.ds(base, EW)], ic)

        g1d = [None] * nfull
        g2d = [None] * nfull
        wbd = [None] * NBUF

        def stage_g1(ci):
            slot = ci % NBUF
            if wbd[slot] is not None:
                wbd[slot].wait()
                wbd[slot] = None
            g1d[ci] = pltpu.async_copy(
                p_hbm.at[ir.at[pl.ds(ci * _CH, _CH)]], ba[slot], gs[slot])

        def stage_g2(ci):
            slot = ci % NBUF
            g1d[ci].wait()
            g2d[ci] = pltpu.async_copy(
                q_hbm.at[ic.at[pl.ds(ci * _CH, _CH)]], ba[slot], hs[slot],
                add=True)

        def stage_wb(ci):
            slot = ci % NBUF
            g2d[ci].wait()
            wbd[slot] = pltpu.async_copy(
                ba[slot], pq_hbm.at[pl.ds(base + ci * _CH, _CH)], ws[slot])

        for ci in range(nfull + 2):
            if ci < nfull:
                stage_g1(ci)
            if 1 <= ci and ci - 1 < nfull:
                stage_g2(ci - 1)
            if 2 <= ci and ci - 2 < nfull:
                stage_wb(ci - 2)
        if rem:
            off = nfull * _CH
            sl = pl.ds(off, rem)
            pltpu.async_copy(p_hbm.at[ir.at[sl]], rba, gs[0]).wait()
            pltpu.async_copy(q_hbm.at[ic.at[sl]], rba, hs[0], add=True).wait()
            pltpu.sync_copy(rba, pq_hbm.at[pl.ds(base + off, rem)])
        for d in wbd:
            if d is not None:
                d.wait()

    return k


def _scatter_call(N, E):
    NW = _NC * _NS
    EW = E // NW
    nfull, rem = EW // _CH, EW % _CH
    NR = -(-N // _NS) // 8 * 8
    NR_LAST = N - (_NS - 1) * NR
    mesh = plsc.VectorSubcoreMesh(core_axis_name="c", subcore_axis_name="s")

    @functools.partial(
        pl.kernel, mesh=mesh,
        out_type=[jax.ShapeDtypeStruct((N, D), jnp.float32)] * _NC,
        scratch_types=[
            [pltpu.VMEM((_CH,), jnp.int32) for _ in range(3)],
            [pltpu.VMEM((_CH, D), jnp.float32) for _ in range(3)],
            pltpu.VMEM((max(rem, 8),), jnp.int32),
            pltpu.VMEM((max(rem, 8), D), jnp.float32),
            pltpu.VMEM_SHARED((N, D), jnp.float32),
            [pltpu.SemaphoreType.DMA for _ in range(3)],
            [pltpu.SemaphoreType.DMA for _ in range(3)],
        ],
    )
    def k(e_hbm, col_hbm, zero_hbm, out0_hbm, out1_hbm, idx2, buf2, ri, rbuf,
          acc, isem, lsem):
        cid = lax.axis_index("c")
        sid = lax.axis_index("s")
        wid = sid * _NC + cid
        base = wid * EW

        def load(ci, slot):
            sl = pl.ds(base + ci * _CH, _CH)
            i = pltpu.async_copy(col_hbm.at[sl], idx2[slot], isem[slot])
            e = pltpu.async_copy(e_hbm.at[sl], buf2[slot], lsem[slot])
            return i, e

        prev = load(0, 0)

        @pl.when(sid < _NS - 1)
        def _():
            pltpu.sync_copy(zero_hbm.at[pl.ds(sid * NR, NR)],
                            acc.at[pl.ds(sid * NR, NR)])

        @pl.when(sid == _NS - 1)
        def _():
            pltpu.sync_copy(zero_hbm.at[pl.ds((_NS - 1) * NR, NR_LAST)],
                            acc.at[pl.ds((_NS - 1) * NR, NR_LAST)])

        plsc.subcore_barrier()

        pend = [prev, load(1, 1)]
        for ci in range(nfull):
            slot = ci % 3
            if ci + 2 < nfull:
                pend.append(load(ci + 2, (ci + 2) % 3))
            cur = pend.pop(0)
            cur[0].wait()
            cur[1].wait()
            pltpu.sync_copy(buf2[slot], acc.at[idx2[slot]], add=True)
        if rem:
            off = nfull * _CH
            pltpu.sync_copy(col_hbm.at[pl.ds(base + off, rem)], ri)
            pltpu.sync_copy(e_hbm.at[pl.ds(base + off, rem)], rbuf)
            pltpu.sync_copy(rbuf, acc.at[ri], add=True)
        plsc.subcore_barrier()
        for c, out_hbm in enumerate((out0_hbm, out1_hbm)):
            @pl.when(jnp.logical_and(cid == c, sid < _NS - 1))
            def _():
                pltpu.sync_copy(acc.at[pl.ds(sid * NR, NR)],
                                out_hbm.at[pl.ds(sid * NR, NR)])

            @pl.when(jnp.logical_and(cid == c, sid == _NS - 1))
            def _():
                pltpu.sync_copy(acc.at[pl.ds((_NS - 1) * NR, NR_LAST)],
                                out_hbm.at[pl.ds((_NS - 1) * NR, NR_LAST)])

    return k


def _ln_res(base, o, g, bb):
    mu = jnp.mean(o, axis=-1, keepdims=True)
    var = jnp.mean((o - mu) ** 2, axis=-1, keepdims=True)
    return base + g * (o - mu) * lax.rsqrt(var + LN_EPS) + bb


def _edge_body(pq_ref, e_ref, w1c, w2, b2, w3, b3, g, bb, out_ref):
    e = e_ref[...]
    h = pq_ref[...] + jnp.dot(e, w1c[...], preferred_element_type=jnp.float32)
    h = jnp.maximum(h, 0.0)
    h = jnp.maximum(jnp.dot(h, w2[...], preferred_element_type=jnp.float32) + b2[...], 0.0)
    o = jnp.dot(h, w3[...], preferred_element_type=jnp.float32) + b3[...]
    out_ref[...] = _ln_res(e, o, g[...], bb[...])


def _node_body(x_ref, a0_ref, a1_ref, w1a, w1b, b1, w2, b2, w3, b3, g, bb,
               p1a, p1b, pb1, x_out, p_out, q_out):
    x = x_ref[...]
    agg = a0_ref[...] + a1_ref[...]
    h = (jnp.dot(x, w1a[...], preferred_element_type=jnp.float32)
         + jnp.dot(agg, w1b[...], preferred_element_type=jnp.float32) + b1[...])
    h = jnp.maximum(h, 0.0)
    h = jnp.maximum(jnp.dot(h, w2[...], preferred_element_type=jnp.float32) + b2[...], 0.0)
    o = jnp.dot(h, w3[...], preferred_element_type=jnp.float32) + b3[...]
    xn = _ln_res(x, o, g[...], bb[...])
    x_out[...] = xn
    p_out[...] = jnp.dot(xn, p1a[...], preferred_element_type=jnp.float32) + pb1[...]
    q_out[...] = jnp.dot(xn, p1b[...], preferred_element_type=jnp.float32)


def _proj_body(x_ref, w1a, w1b, b1, p_out, q_out):
    x = x_ref[...]
    p_out[...] = jnp.dot(x, w1a[...], preferred_element_type=jnp.float32) + b1[...]
    q_out[...] = jnp.dot(x, w1b[...], preferred_element_type=jnp.float32)


def _full(shape):
    return pl.BlockSpec(shape, lambda i: (0,) * len(shape))


def _rows(block):
    return pl.BlockSpec((block, D), lambda i: (i, 0))


def _edge_call(E, BE):
    grid = E // BE
    w = _full((D, D))
    v = _full((1, D))
    return pl.pallas_call(
        _edge_body,
        grid=(grid,),
        in_specs=[_rows(BE), _rows(BE), w, w, v, w, v, v, v],
        out_specs=_rows(BE),
        out_shape=jax.ShapeDtypeStruct((E, D), jnp.float32),
    )


def _node_call(N, BN):
    grid = N // BN
    w = _full((D, D))
    v = _full((1, D))
    out = jax.ShapeDtypeStruct((N, D), jnp.float32)
    outh = jax.ShapeDtypeStruct((N, D), jnp.bfloat16)
    return pl.pallas_call(
        _node_body,
        grid=(grid,),
        in_specs=[_rows(BN), _rows(BN), _rows(BN), w, w, v, w, v, w, v, v, v, w, w, v],
        out_specs=[_rows(BN), _rows(BN), _rows(BN)],
        out_shape=[out, out, out],
    )


def _proj_call(N, BN):
    grid = N // BN
    w = _full((D, D))
    v = _full((1, D))
    out = jax.ShapeDtypeStruct((N, D), jnp.float32)
    return pl.pallas_call(
        _proj_body,
        grid=(grid,),
        in_specs=[_rows(BN), w, w, v],
        out_specs=[_rows(BN), _rows(BN)],
        out_shape=[out, out],
    )


def kernel(x, edge_indices, edge_attrs, We1, be1, We2, be2, We3, be3, eg, eb,
           Wn1, bn1, Wn2, bn2, Wn3, bn3, ng, nb):
    N, _ = x.shape
    E = edge_attrs.shape[1]
    T = We1.shape[0]
    row = edge_indices[0, 0]
    col = edge_indices[0, 1]
    e = edge_attrs[0]

    BE, BN = 10000, 2000
    edge_fn = _edge_call(E, BE)
    node_fn = _node_call(N, BN)
    proj_fn = _proj_call(N, BN)
    gather_fn = _gather_call(N, E)
    scatter_fn = _scatter_call(N, E)

    r2 = lambda a: a.reshape(1, D)
    zeros_nd = jnp.zeros((N, D), jnp.float32)

    p, q = proj_fn(x, We1[0, :D], We1[0, D:2 * D], r2(be1[0]))
    for t in range(T):
        pq = gather_fn(p, q, row, col)
        e = edge_fn(pq, e, We1[t, 2 * D:], We2[t], r2(be2[t]), We3[t],
                    r2(be3[t]), r2(eg[t]), r2(eb[t]))
        a0, a1 = scatter_fn(e, col, zeros_nd)
        tn = (t + 1) % T
        x, p, q = node_fn(x, a0, a1, Wn1[t, :D], Wn1[t, D:],
                          r2(bn1[t]), Wn2[t], r2(bn2[t]), Wn3[t], r2(bn3[t]),
                          r2(ng[t]), r2(nb[t]),
                          We1[tn, :D], We1[tn, D:2 * D], r2(be1[tn]))
    return (x, e)
